# baseline (speedup 1.0000x reference)
import jax
import jax.numpy as jnp
from jax import lax
from jax.experimental import pallas as pl
from jax.experimental.pallas import tpu as pltpu
from jax.experimental.pallas import tpu_sc as plsc

N_NODES = 10000
N_EDGES = 160000
D = 256
DH = 128

NS = 16
EPT = N_EDGES // NS
KC = 80
NCH = EPT // KC
NPAD = 10240
RPT = NPAD // NS


def _tc_a_body(x_ref, w_ref, asrc_ref, adst_ref,
               e0_ref, e1_ref, e2_ref, e3_ref, ae1_ref, ae2_ref,
               hcat_ref, s_ref, d_ref, g1_ref, g2_ref):
    h = jnp.dot(x_ref[...], w_ref[...], preferred_element_type=jnp.float32)
    hcat_ref[0] = h[:, :DH]
    hcat_ref[1] = h[:, DH:]
    s_ref[...] = jnp.dot(h, asrc_ref[...], preferred_element_type=jnp.float32)
    d_ref[...] = jnp.dot(h, adst_ref[...], preferred_element_type=jnp.float32)
    e0 = e0_ref[...]
    e1 = e1_ref[...]
    e2 = e2_ref[...]
    e3 = e3_ref[...]
    g1_ref[...] = (e0 * ae1_ref[0] + e1 * ae1_ref[1]
                   + e2 * ae1_ref[2] + e3 * ae1_ref[3])
    g2_ref[...] = (e0 * ae2_ref[0] + e1 * ae2_ref[1]
                   + e2 * ae2_ref[2] + e3 * ae2_ref[3])


_tc_a = pl.pallas_call(
    _tc_a_body,
    out_shape=(
        jax.ShapeDtypeStruct((2, N_NODES, DH), jnp.float32),
        jax.ShapeDtypeStruct((N_NODES,), jnp.float32),
        jax.ShapeDtypeStruct((N_NODES,), jnp.float32),
        jax.ShapeDtypeStruct((N_EDGES,), jnp.float32),
        jax.ShapeDtypeStruct((N_EDGES,), jnp.float32),
    ),
    in_specs=[pl.BlockSpec(memory_space=pltpu.VMEM)] * 8
    + [pl.BlockSpec(memory_space=pltpu.SMEM)] * 2,
)


def _tc_b_body(xcat_ref, w_ref, asrc_ref, adst_ref, hcat_ref, s_ref, d_ref):
    h = (jnp.dot(xcat_ref[0], w_ref[:DH, :], preferred_element_type=jnp.float32)
         + jnp.dot(xcat_ref[1], w_ref[DH:, :],
                   preferred_element_type=jnp.float32))
    hcat_ref[0] = h[:, :DH]
    hcat_ref[1] = h[:, DH:]
    s_ref[...] = jnp.dot(h, asrc_ref[...], preferred_element_type=jnp.float32)
    d_ref[...] = jnp.dot(h, adst_ref[...], preferred_element_type=jnp.float32)


_tc_b = pl.pallas_call(
    _tc_b_body,
    out_shape=(
        jax.ShapeDtypeStruct((2, N_NODES, DH), jnp.float32),
        jax.ShapeDtypeStruct((N_NODES,), jnp.float32),
        jax.ShapeDtypeStruct((N_NODES,), jnp.float32),
    ),
)


def _splat16(val):
    return jnp.full((16,), val, dtype=jnp.int32)


def _make_sc(first_layer):
    if first_layer:
        out_type = (
            jax.ShapeDtypeStruct((2, NPAD, DH), jnp.float32),
            jax.ShapeDtypeStruct((N_EDGES,), jnp.float32),
            jax.ShapeDtypeStruct((NPAD,), jnp.float32),
        )
    else:
        out_type = jax.ShapeDtypeStruct((2, NPAD, DH), jnp.float32)

    mesh = plsc.VectorSubcoreMesh(core_axis_name="c", subcore_axis_name="s")

    def body(*refs):
        if first_layer:
            (hcat, src1, dst1, g1d, s1d, d1d,
             out_hbm, p_hbm, den_hbm,
             s_v, d_v, den1_v, sb, db, gb, pb, ab,
             gidx_v, rows_v, denb_v, sem, den_sh, acc_sh) = refs
        else:
            (hcat, src1, dst1, g1d, s1d, d1d, p1_1d, den1_hbm,
             out_hbm,
             s_v, d_v, den1_v, sb, db, gb, pb, ab,
             gidx_v, rows_v, denb_v, sem, den_sh, acc_sh) = refs
        c = lax.axis_index("c")
        s = lax.axis_index("s")
        ebase = s * EPT

        pltpu.sync_copy(s1d, s_v)
        pltpu.sync_copy(d1d, d_v)
        if not first_layer:
            pltpu.sync_copy(den1_hbm, den1_v)

        def rows_zero(r):
            for cb in range(DH // 16):
                rows_v[r, pl.ds(cb * 16, 16)] = jnp.zeros((16,), jnp.float32)
        pl.loop(0, KC, unroll=4)(rows_zero)

        def denb_zero(i):
            denb_v[pl.ds(i * 16, 16)] = jnp.zeros((16,), jnp.float32)
        pl.loop(0, RPT // 16)(denb_zero)

        pltpu.sync_copy(denb_v, den_sh.at[pl.ds(s * RPT, RPT)])
        for k in range(RPT // KC):
            pltpu.sync_copy(rows_v, acc_sh.at[pl.ds(s * RPT + k * KC, KC)])
        plsc.subcore_barrier()

        def chunk(j):
            cbase = ebase + j * KC
            pltpu.sync_copy(src1.at[pl.ds(cbase, KC)], sb.at[0])
            pltpu.sync_copy(dst1.at[pl.ds(cbase, KC)], db.at[0])
            pltpu.sync_copy(g1d.at[pl.ds(cbase, KC)], gb.at[0])
            if not first_layer:
                pltpu.sync_copy(p1_1d.at[pl.ds(cbase, KC)], ab.at[0])

            def qstep(q):
                sl = pl.ds(q * 16, 16)
                si = sb[0, sl]
                di = db[0, sl]
                sv = plsc.load_gather(s_v, [si])
                dv = plsc.load_gather(d_v, [di])
                g = gb[0, sl]
                if not first_layer:
                    dn1 = plsc.load_gather(den1_v, [di])
                    g = g * ab[0, sl] / (dn1 + 1e-16)
                t = sv + dv + g
                t = jnp.where(t >= 0.0, t, 0.2 * t)
                pb[0, sl] = jnp.exp(t)
            pl.loop(0, KC // 16, unroll=5)(qstep)

            pltpu.sync_copy(pb.at[0], den_sh.at[db.at[0]], add=True)
            if first_layer:
                @pl.when(c == 0)
                def _():
                    pltpu.sync_copy(pb.at[0], p_hbm.at[pl.ds(cbase, KC)])

            hbase = c * N_NODES

            def gi_step(q):
                sl = pl.ds(q * 16, 16)
                gidx_v[sl] = sb[0, sl] + hbase
            pl.loop(0, KC // 16, unroll=5)(gi_step)
            pltpu.async_copy(hcat.at[gidx_v], rows_v, sem).wait()

            def scale(rq):
                for r16 in range(16):
                    spl = plsc.load_gather(
                        pb, [_splat16(0), _splat16(rq * 16 + r16)])
                    r = rq * 16 + r16
                    for cb in range(DH // 16):
                        sl = pl.ds(cb * 16, 16)
                        rows_v[r, sl] = rows_v[r, sl] * spl
            pl.loop(0, KC // 16, unroll=5)(scale)

            pltpu.sync_copy(rows_v, acc_sh.at[db.at[0]], add=True)
        pl.loop(0, NCH)(chunk)
        plsc.subcore_barrier()

        pltpu.sync_copy(den_sh.at[pl.ds(s * RPT, RPT)], denb_v)
        if first_layer:
            @pl.when(c == 0)
            def _():
                pltpu.sync_copy(denb_v, den_hbm.at[pl.ds(s * RPT, RPT)])

        def wchunk(k):
            rbase = s * RPT + k * KC
            pltpu.sync_copy(acc_sh.at[pl.ds(rbase, KC)], rows_v)

            def wscale(rq):
                for r16 in range(16):
                    r = rq * 16 + r16
                    dn = plsc.load_gather(denb_v, [_splat16(k * KC + r)])
                    inv = 1.0 / (dn + 1e-16)
                    for cb in range(DH // 16):
                        sl = pl.ds(cb * 16, 16)
                        rows_v[r, sl] = rows_v[r, sl] * inv
            pl.loop(0, KC // 16, unroll=5)(wscale)
            pltpu.sync_copy(rows_v, out_hbm.at[c, pl.ds(rbase, KC)])
        pl.loop(0, RPT // KC)(wchunk)

    scratch = [
        pltpu.VMEM((N_NODES,), jnp.float32),
        pltpu.VMEM((N_NODES,), jnp.float32),
        pltpu.VMEM((16 if first_layer else N_NODES,), jnp.float32),
        pltpu.VMEM((1, KC), jnp.int32),
        pltpu.VMEM((1, KC), jnp.int32),
        pltpu.VMEM((1, KC), jnp.float32),
        pltpu.VMEM((1, KC), jnp.float32),
        pltpu.VMEM((1, KC), jnp.float32),
        pltpu.VMEM((KC,), jnp.int32),
        pltpu.VMEM((KC, DH), jnp.float32),
        pltpu.VMEM((RPT,), jnp.float32),
        pltpu.SemaphoreType.DMA,
        pltpu.VMEM_SHARED((NPAD,), jnp.float32),
        pltpu.VMEM_SHARED((NPAD, DH), jnp.float32),
    ]
    return pl.kernel(
        body, out_type=out_type, mesh=mesh, scratch_types=scratch,
        compiler_params=pltpu.CompilerParams(needs_layout_passes=False))


_sc_layer1 = _make_sc(first_layer=True)
_sc_layer2 = _make_sc(first_layer=False)


@jax.jit
def _impl(x, edge_index, edge_attr, W1, a_src1, a_dst1, a_edge1,
          W2, a_src2, a_dst2, a_edge2):
    src1 = edge_index[0]
    dst1 = edge_index[1]
    ecols = [jnp.asarray(edge_attr[:, k]) for k in range(4)]

    hcat1, s1, d1, g1, g2base = _tc_a(
        x, W1, a_src1, a_dst1, ecols[0], ecols[1], ecols[2], ecols[3],
        a_edge1, a_edge2)

    x1pad, p1, den1 = _sc_layer1(
        hcat1.reshape(2 * N_NODES, DH), src1, dst1, g1, s1, d1)

    hcat2, s2, d2 = _tc_b(x1pad[:, :N_NODES, :], W2, a_src2, a_dst2)

    x2pad = _sc_layer2(
        hcat2.reshape(2 * N_NODES, DH), src1, dst1, g2base, s2, d2,
        p1, den1[:N_NODES])

    return jnp.concatenate([x2pad[0, :N_NODES, :], x2pad[1, :N_NODES, :]],
                           axis=1)


def kernel(x, edge_index, edge_attr, W1, a_src1, a_dst1, a_edge1,
           W2, a_src2, a_dst2, a_edge2):
    return _impl(x, edge_index, edge_attr, W1, a_src1, a_dst1, a_edge1,
                 W2, a_src2, a_dst2, a_edge2)


# speedup vs baseline: 1.3865x; 1.3865x over previous
import jax
import jax.numpy as jnp
from jax import lax
from jax.experimental import pallas as pl
from jax.experimental.pallas import tpu as pltpu
from jax.experimental.pallas import tpu_sc as plsc

N_NODES = 10000
N_EDGES = 160000
D = 256
DH = 128

NS = 16
EPT = N_EDGES // NS
KC = 80
NCH = EPT // KC
NPAD = 10240
RPT = NPAD // NS

_MESH = plsc.VectorSubcoreMesh(core_axis_name="c", subcore_axis_name="s")
_SC_PARAMS = pltpu.CompilerParams(needs_layout_passes=False)


def _tc_a_body(x_ref, w_ref, asrc_ref, adst_ref,
               e0_ref, e1_ref, e2_ref, e3_ref, ae1_ref, ae2_ref,
               hcat_ref, s_ref, d_ref, g1_ref, g2_ref):
    h = jnp.dot(x_ref[...], w_ref[...], preferred_element_type=jnp.float32)
    hcat_ref[0] = h[:, :DH]
    hcat_ref[1] = h[:, DH:]
    s_ref[...] = jnp.dot(h, asrc_ref[...], preferred_element_type=jnp.float32)
    d_ref[...] = jnp.dot(h, adst_ref[...], preferred_element_type=jnp.float32)
    e0 = e0_ref[...]
    e1 = e1_ref[...]
    e2 = e2_ref[...]
    e3 = e3_ref[...]
    g1_ref[...] = (e0 * ae1_ref[0] + e1 * ae1_ref[1]
                   + e2 * ae1_ref[2] + e3 * ae1_ref[3])
    g2_ref[...] = (e0 * ae2_ref[0] + e1 * ae2_ref[1]
                   + e2 * ae2_ref[2] + e3 * ae2_ref[3])


_tc_a = pl.pallas_call(
    _tc_a_body,
    out_shape=(
        jax.ShapeDtypeStruct((2, N_NODES, DH), jnp.float32),
        jax.ShapeDtypeStruct((N_NODES,), jnp.float32),
        jax.ShapeDtypeStruct((N_NODES,), jnp.float32),
        jax.ShapeDtypeStruct((N_EDGES,), jnp.float32),
        jax.ShapeDtypeStruct((N_EDGES,), jnp.float32),
    ),
    in_specs=[pl.BlockSpec(memory_space=pltpu.VMEM)] * 8
    + [pl.BlockSpec(memory_space=pltpu.SMEM)] * 2,
)


def _tc_b_body(xcat_ref, w_ref, asrc_ref, adst_ref, al_ref, g2b_ref,
               hcat_ref, s_ref, d_ref, g2_ref):
    h = (jnp.dot(xcat_ref[0], w_ref[:DH, :], preferred_element_type=jnp.float32)
         + jnp.dot(xcat_ref[1], w_ref[DH:, :],
                   preferred_element_type=jnp.float32))
    hcat_ref[0] = h[:, :DH]
    hcat_ref[1] = h[:, DH:]
    s_ref[...] = jnp.dot(h, asrc_ref[...], preferred_element_type=jnp.float32)
    d_ref[...] = jnp.dot(h, adst_ref[...], preferred_element_type=jnp.float32)
    g2_ref[...] = al_ref[...] * g2b_ref[...]


_tc_b = pl.pallas_call(
    _tc_b_body,
    out_shape=(
        jax.ShapeDtypeStruct((2, N_NODES, DH), jnp.float32),
        jax.ShapeDtypeStruct((N_NODES,), jnp.float32),
        jax.ShapeDtypeStruct((N_NODES,), jnp.float32),
        jax.ShapeDtypeStruct((N_EDGES,), jnp.float32),
    ),
)


def _splat16(val):
    return jnp.full((16,), val, dtype=jnp.int32)


def _alpha_body(dst1, p1, den1, alpha_out, dst_v, p_v, den_v):
    c = lax.axis_index("c")
    s = lax.axis_index("s")

    @pl.when(c == 0)
    def _():
        ebase = s * EPT
        pltpu.sync_copy(dst1.at[pl.ds(ebase, EPT)], dst_v)
        pltpu.sync_copy(p1.at[pl.ds(ebase, EPT)], p_v)
        pltpu.sync_copy(den1, den_v)

        def step(q):
            sl = pl.ds(q * 16, 16)
            di = dst_v[sl]
            dn = plsc.load_gather(den_v, [di])
            p_v[sl] = p_v[sl] / (dn + 1e-16)
        pl.loop(0, EPT // 16)(step)
        pltpu.sync_copy(p_v, alpha_out.at[pl.ds(ebase, EPT)])


_sc_alpha = pl.kernel(
    _alpha_body,
    out_type=jax.ShapeDtypeStruct((N_EDGES,), jnp.float32),
    mesh=_MESH,
    scratch_types=[
        pltpu.VMEM((EPT,), jnp.int32),
        pltpu.VMEM((EPT,), jnp.float32),
        pltpu.VMEM((N_NODES,), jnp.float32),
    ],
    compiler_params=_SC_PARAMS,
)


def _make_sc(first_layer):
    if first_layer:
        out_type = (
            jax.ShapeDtypeStruct((2, NPAD, DH), jnp.float32),
            jax.ShapeDtypeStruct((N_EDGES,), jnp.float32),
            jax.ShapeDtypeStruct((NPAD,), jnp.float32),
        )
    else:
        out_type = jax.ShapeDtypeStruct((2, NPAD, DH), jnp.float32)

    def body(*refs):
        if first_layer:
            (hcat, src1, dst1, g1d, s1d, d1d,
             out_hbm, p_hbm, den_hbm,
             s_v, d_v, sb, db, gb, pb, gidx2, rows_v, denb_v,
             gsem0, gsem1, ssem0, ssem1, sem, den_sh, acc_sh) = refs
        else:
            (hcat, src1, dst1, g1d, s1d, d1d,
             out_hbm,
             s_v, d_v, sb, db, gb, pb, gidx2, rows_v, denb_v,
             gsem0, gsem1, ssem0, ssem1, sem, den_sh, acc_sh) = refs
            p_hbm = den_hbm = None
        c = lax.axis_index("c")
        s = lax.axis_index("s")
        ebase = s * EPT
        hbase = c * N_NODES
        gsem = (gsem0, gsem1)
        ssem = (ssem0, ssem1)

        pltpu.sync_copy(s1d, s_v)
        pltpu.sync_copy(d1d, d_v)

        def rows_zero(r):
            for cb in range(DH // 16):
                rows_v[0, r, pl.ds(cb * 16, 16)] = jnp.zeros((16,),
                                                             jnp.float32)
        pl.loop(0, KC)(rows_zero)

        def denb_zero(i):
            denb_v[pl.ds(i * 16, 16)] = jnp.zeros((16,), jnp.float32)
        pl.loop(0, RPT // 16)(denb_zero)

        pltpu.sync_copy(denb_v, den_sh.at[pl.ds(s * RPT, RPT)])
        for k in range(RPT // KC):
            pltpu.sync_copy(rows_v.at[0],
                            acc_sh.at[pl.ds(s * RPT + k * KC, KC)])
        plsc.subcore_barrier()

        def prep(j, b):
            cbase = ebase + j * KC
            pltpu.sync_copy(src1.at[pl.ds(cbase, KC)], sb.at[0])
            pltpu.sync_copy(dst1.at[pl.ds(cbase, KC)], db.at[b])
            pltpu.sync_copy(g1d.at[pl.ds(cbase, KC)], gb.at[0])

            def qstep(q):
                sl = pl.ds(q * 16, 16)
                sv = plsc.load_gather(s_v, [sb[0, sl]])
                dv = plsc.load_gather(d_v, [db[b, sl]])
                t = sv + dv + gb[0, sl]
                t = jnp.where(t >= 0.0, t, 0.2 * t)
                pb[b, sl] = jnp.exp(t)
            pl.loop(0, KC // 16)(qstep)

            pltpu.sync_copy(pb.at[b], den_sh.at[db.at[b]], add=True)
            if first_layer:
                @pl.when(c == 0)
                def _():
                    pltpu.sync_copy(pb.at[b], p_hbm.at[pl.ds(cbase, KC)])

            def gi_step(q):
                sl = pl.ds(q * 16, 16)
                gidx2[b, sl] = sb[0, sl] + hbase
            pl.loop(0, KC // 16)(gi_step)

        def fire_gather(b):
            pltpu.async_copy(hcat.at[gidx2.at[b]], rows_v.at[b], gsem[b])

        def wait_gather(b):
            pltpu.make_async_copy(hcat.at[gidx2.at[b]], rows_v.at[b],
                                  gsem[b]).wait()

        def scale(b):
            def srq(rq):
                for r16 in range(16):
                    spl = plsc.load_gather(
                        pb, [_splat16(b), _splat16(rq * 16 + r16)])
                    r = rq * 16 + r16
                    for cb in range(DH // 16):
                        sl = pl.ds(cb * 16, 16)
                        rows_v[b, r, sl] = rows_v[b, r, sl] * spl
            pl.loop(0, KC // 16)(srq)

        def fire_scatter(b):
            pltpu.async_copy(rows_v.at[b], acc_sh.at[db.at[b]], ssem[b],
                             add=True)

        def drain_scatter(b):
            pltpu.make_async_copy(rows_v.at[b], acc_sh.at[db.at[b]],
                                  ssem[b]).wait()

        prep(0, 0)
        fire_gather(0)

        def pair(t):
            @pl.when(t > 0)
            def _():
                drain_scatter(1)
            prep(2 * t + 1, 1)
            fire_gather(1)
            wait_gather(0)
            scale(0)
            fire_scatter(0)
            drain_scatter(0)
            prep(2 * t + 2, 0)
            fire_gather(0)
            wait_gather(1)
            scale(1)
            fire_scatter(1)
        pl.loop(0, (NCH - 1) // 2)(pair)

        drain_scatter(1)
        wait_gather(0)
        scale(0)
        fire_scatter(0)
        drain_scatter(0)
        plsc.subcore_barrier()

        pltpu.sync_copy(den_sh.at[pl.ds(s * RPT, RPT)], denb_v)
        if first_layer:
            @pl.when(c == 0)
            def _():
                pltpu.sync_copy(denb_v, den_hbm.at[pl.ds(s * RPT, RPT)])

        def wchunk(k):
            rbase = s * RPT + k * KC
            pltpu.sync_copy(acc_sh.at[pl.ds(rbase, KC)], rows_v.at[0])

            def wscale(rq):
                for r16 in range(16):
                    r = rq * 16 + r16
                    dn = plsc.load_gather(denb_v, [_splat16(k * KC + r)])
                    inv = 1.0 / (dn + 1e-16)
                    for cb in range(DH // 16):
                        sl = pl.ds(cb * 16, 16)
                        rows_v[0, r, sl] = rows_v[0, r, sl] * inv
            pl.loop(0, KC // 16)(wscale)
            pltpu.sync_copy(rows_v.at[0], out_hbm.at[c, pl.ds(rbase, KC)])
        pl.loop(0, RPT // KC)(wchunk)

    scratch = [
        pltpu.VMEM((N_NODES,), jnp.float32),
        pltpu.VMEM((N_NODES,), jnp.float32),
        pltpu.VMEM((1, KC), jnp.int32),
        pltpu.VMEM((2, KC), jnp.int32),
        pltpu.VMEM((1, KC), jnp.float32),
        pltpu.VMEM((2, KC), jnp.float32),
        pltpu.VMEM((2, KC), jnp.int32),
        pltpu.VMEM((2, KC, DH), jnp.float32),
        pltpu.VMEM((RPT,), jnp.float32),
        pltpu.SemaphoreType.DMA,
        pltpu.SemaphoreType.DMA,
        pltpu.SemaphoreType.DMA,
        pltpu.SemaphoreType.DMA,
        pltpu.SemaphoreType.DMA,
        pltpu.VMEM_SHARED((NPAD,), jnp.float32),
        pltpu.VMEM_SHARED((NPAD, DH), jnp.float32),
    ]
    return pl.kernel(body, out_type=out_type, mesh=_MESH,
                     scratch_types=scratch, compiler_params=_SC_PARAMS)


_sc_layer1 = _make_sc(first_layer=True)
_sc_layer2 = _make_sc(first_layer=False)


@jax.jit
def _impl(x, edge_index, edge_attr, W1, a_src1, a_dst1, a_edge1,
          W2, a_src2, a_dst2, a_edge2):
    src1 = edge_index[0]
    dst1 = edge_index[1]
    ecols = [jnp.asarray(edge_attr[:, k]) for k in range(4)]

    hcat1, s1, d1, g1, g2base = _tc_a(
        x, W1, a_src1, a_dst1, ecols[0], ecols[1], ecols[2], ecols[3],
        a_edge1, a_edge2)

    x1pad, p1, den1 = _sc_layer1(
        hcat1.reshape(2 * N_NODES, DH), src1, dst1, g1, s1, d1)

    alpha1 = _sc_alpha(dst1, p1, den1[:N_NODES])

    hcat2, s2, d2, g2 = _tc_b(x1pad[:, :N_NODES, :], W2, a_src2, a_dst2,
                              alpha1, g2base)

    x2pad = _sc_layer2(
        hcat2.reshape(2 * N_NODES, DH), src1, dst1, g2, s2, d2)

    return jnp.concatenate([x2pad[0, :N_NODES, :], x2pad[1, :N_NODES, :]],
                           axis=1)


def kernel(x, edge_index, edge_attr, W1, a_src1, a_dst1, a_edge1,
           W2, a_src2, a_dst2, a_edge2):
    return _impl(x, edge_index, edge_attr, W1, a_src1, a_dst1, a_edge1,
                 W2, a_src2, a_dst2, a_edge2)
